# baseline (speedup 1.0000x reference)
import jax
import jax.numpy as jnp
from jax.experimental import pallas as pl
from jax.experimental.pallas import tpu as pltpu

_BM = 600


def _body(x_ref, adj_ref, w1_ref, b1_ref, w2_ref, b2_ref, out_ref, h_ref):
    layer = pl.program_id(0)
    i = pl.program_id(1)
    n = x_ref.shape[0]

    @pl.when(layer == 0)
    def _():
        acc = jnp.dot(adj_ref[...], x_ref[...], preferred_element_type=jnp.float32)
        h = jnp.dot(acc, w1_ref[...], preferred_element_type=jnp.float32) + b1_ref[...]
        h_ref[pl.ds(i * _BM, _BM), :] = jnp.maximum(h, 0.0)

    @pl.when(layer == 1)
    def _():
        acc = jnp.dot(adj_ref[...], h_ref[:n, :], preferred_element_type=jnp.float32)
        h = jnp.dot(acc, w2_ref[...], preferred_element_type=jnp.float32) + b2_ref[...]
        out_ref[...] = jnp.maximum(h, 0.0)


def kernel(x, adj, W1, b1, W2, b2):
    n, f = x.shape
    nblocks = pl.cdiv(n, _BM)
    const = lambda l, i: (0, 0)
    return pl.pallas_call(
        _body,
        grid=(2, nblocks),
        in_specs=[
            pl.BlockSpec((n, f), const),
            pl.BlockSpec((_BM, n), lambda l, i: (i, 0)),
            pl.BlockSpec((f, f), const),
            pl.BlockSpec((1, f), const),
            pl.BlockSpec((f, f), const),
            pl.BlockSpec((1, f), const),
        ],
        out_specs=pl.BlockSpec((_BM, f), lambda l, i: (i, 0)),
        out_shape=jax.ShapeDtypeStruct((n, f), jnp.float32),
        scratch_shapes=[pltpu.VMEM((nblocks * _BM, f), jnp.float32)],
    )(x, adj, W1, b1.reshape(1, f), W2, b2.reshape(1, f))


# speedup vs baseline: 1.0106x; 1.0106x over previous
import jax
import jax.numpy as jnp
from jax.experimental import pallas as pl
from jax.experimental.pallas import tpu as pltpu

_BM = 480


def _body(x_ref, adj_ref, w1_ref, b1_ref, w2_ref, b2_ref, out_ref, h_ref):
    layer = pl.program_id(0)
    i = pl.program_id(1)
    n = x_ref.shape[0]

    @pl.when(layer == 0)
    def _():
        acc = jnp.dot(adj_ref[...], x_ref[...], preferred_element_type=jnp.float32)
        h = jnp.dot(acc, w1_ref[...], preferred_element_type=jnp.float32) + b1_ref[...]
        h_ref[pl.ds(i * _BM, _BM), :] = jnp.maximum(h, 0.0)

    @pl.when(layer == 1)
    def _():
        acc = jnp.dot(adj_ref[...], h_ref[:n, :], preferred_element_type=jnp.float32)
        h = jnp.dot(acc, w2_ref[...], preferred_element_type=jnp.float32) + b2_ref[...]
        out_ref[...] = jnp.maximum(h, 0.0)


def kernel(x, adj, W1, b1, W2, b2):
    n, f = x.shape
    nblocks = pl.cdiv(n, _BM)
    const = lambda l, i: (0, 0)
    return pl.pallas_call(
        _body,
        grid=(2, nblocks),
        in_specs=[
            pl.BlockSpec((n, f), const),
            pl.BlockSpec((_BM, n), lambda l, i: (i, 0)),
            pl.BlockSpec((f, f), const),
            pl.BlockSpec((1, f), const),
            pl.BlockSpec((f, f), const),
            pl.BlockSpec((1, f), const),
        ],
        out_specs=pl.BlockSpec((_BM, f), lambda l, i: (i, 0)),
        out_shape=jax.ShapeDtypeStruct((n, f), jnp.float32),
        scratch_shapes=[pltpu.VMEM((nblocks * _BM, f), jnp.float32)],
    )(x, adj, W1, b1.reshape(1, f), W2, b2.reshape(1, f))


# speedup vs baseline: 1.0218x; 1.0110x over previous
import jax
import jax.numpy as jnp
from jax.experimental import pallas as pl
from jax.experimental.pallas import tpu as pltpu

_BM = 400


def _body(x_ref, adj_ref, w1_ref, b1_ref, w2_ref, b2_ref, out_ref, h_ref):
    layer = pl.program_id(0)
    i = pl.program_id(1)
    n = x_ref.shape[0]

    @pl.when(layer == 0)
    def _():
        acc = jnp.dot(adj_ref[...], x_ref[...], preferred_element_type=jnp.float32)
        h = jnp.dot(acc, w1_ref[...], preferred_element_type=jnp.float32) + b1_ref[...]
        h_ref[pl.ds(i * _BM, _BM), :] = jnp.maximum(h, 0.0)

    @pl.when(layer == 1)
    def _():
        acc = jnp.dot(adj_ref[...], h_ref[:n, :], preferred_element_type=jnp.float32)
        h = jnp.dot(acc, w2_ref[...], preferred_element_type=jnp.float32) + b2_ref[...]
        out_ref[...] = jnp.maximum(h, 0.0)


def kernel(x, adj, W1, b1, W2, b2):
    n, f = x.shape
    nblocks = pl.cdiv(n, _BM)
    const = lambda l, i: (0, 0)
    return pl.pallas_call(
        _body,
        grid=(2, nblocks),
        in_specs=[
            pl.BlockSpec((n, f), const),
            pl.BlockSpec((_BM, n), lambda l, i: (i, 0)),
            pl.BlockSpec((f, f), const),
            pl.BlockSpec((1, f), const),
            pl.BlockSpec((f, f), const),
            pl.BlockSpec((1, f), const),
        ],
        out_specs=pl.BlockSpec((_BM, f), lambda l, i: (i * l, 0)),
        out_shape=jax.ShapeDtypeStruct((n, f), jnp.float32),
        scratch_shapes=[pltpu.VMEM((nblocks * _BM, f), jnp.float32)],
    )(x, adj, W1, b1.reshape(1, f), W2, b2.reshape(1, f))
